# Initial kernel scaffold; baseline (speedup 1.0000x reference)
import jax
import jax.numpy as jnp
from jax.experimental import pallas as pl


def kernel(mem, idx, val, sample_idx):
    raise NotImplementedError("write your pallas kernel here")


# speedup vs baseline: 1.9939x; 1.9939x over previous
import jax
import jax.numpy as jnp
from jax import lax
from jax.experimental import pallas as pl
from jax.experimental.pallas import tpu as pltpu
from jax.experimental.pallas import tpu_sc as plsc

M = 1000000
B = 16384
D = 64
L = 16
NC = 2
NS = 16
MPAD = 1 << 20
RPT = MPAD // NS
JBITS = 14
SPW = B // (NC * NS)
CH = 128
NCH = SPW // CH
INVALID = 0x7FFFFFFF


def _body(mem_hbm, idx_hbm, val_hbm, samp_hbm, out_hbm,
          idx_v, tag_v, samp_v, sadj_v, t_v, tc_v, mf_v, g_v, v_v, tag_hbm):
    cid = lax.axis_index("c")
    sid = lax.axis_index("s")
    lanes = lax.iota(jnp.int32, L)
    shift = jnp.minimum(lanes + 1, L - 1)

    pltpu.sync_copy(idx_hbm, idx_v)
    neg1 = jnp.full((L,), -1, jnp.int32)

    def init_body(i, _):
        tag_v[pl.ds(i * L, L)] = neg1
        return _

    lax.fori_loop(0, RPT // L, init_body, None)

    base_row = sid * RPT

    def scan_body(k, _):
        x = idx_v[pl.ds(k * L, L)]
        jv = k * L + lanes
        local = x - base_row
        valid = (local >= 0) & (local < RPT)
        comp = jnp.where(valid, (local << JBITS) | jv, INVALID)
        comp_s, _unused_vals = plsc.sort_key_val(comp, comp)
        loc_s = lax.shift_right_arithmetic(comp_s, JBITS)
        j_s = comp_s & (B - 1)
        valid_s = comp_s < (1 << (JBITS + 16))
        nxt = comp_s.at[shift].get(mode="promise_in_bounds")
        nxt_loc = lax.shift_right_arithmetic(nxt, JBITS)
        win = valid_s & ((loc_s != nxt_loc) | (lanes == L - 1))
        loc_c = jnp.minimum(loc_s, RPT - 1)
        plsc.store_scatter(tag_v, [loc_c], j_s, mask=win)
        return _

    lax.fori_loop(0, B // L, scan_body, None)

    pltpu.sync_copy(tag_v, tag_hbm.at[pl.ds(cid * MPAD + sid * RPT, RPT)])
    plsc.subcore_barrier()

    base_s = (cid * NS + sid) * SPW
    pltpu.sync_copy(samp_hbm.at[pl.ds(base_s, SPW)], samp_v)

    def adj_body(i, _):
        sadj_v[pl.ds(i * L, L)] = samp_v[pl.ds(i * L, L)] + cid * MPAD
        return _

    lax.fori_loop(0, SPW // L, adj_body, None)

    def chunk_body(c, _):
        sl = samp_v.at[pl.ds(c * CH, CH)]
        sla = sadj_v.at[pl.ds(c * CH, CH)]
        pltpu.sync_copy(tag_hbm.at[sla], t_v)

        def mask_body(i, _):
            t = t_v[pl.ds(i * L, L)]
            hit = t >= 0
            tc_v[pl.ds(i * L, L)] = jnp.where(hit, t, 0)
            mf_v[pl.ds(i * L, L)] = jnp.where(hit, 1.0, 0.0).astype(jnp.float32)
            return _

        lax.fori_loop(0, CH // L, mask_body, None)
        pltpu.sync_copy(mem_hbm.at[sl], g_v)
        pltpu.sync_copy(val_hbm.at[tc_v], v_v)

        def row_body(r, _):
            mrow = plsc.load_gather(mf_v, [jnp.full((L,), r, jnp.int32)])
            for cc in range(D // L):
                g = g_v[r, pl.ds(cc * L, L)]
                v = v_v[r, pl.ds(cc * L, L)]
                g_v[r, pl.ds(cc * L, L)] = g + mrow * (v - g)
            return _

        lax.fori_loop(0, CH, row_body, None)
        pltpu.sync_copy(g_v, out_hbm.at[pl.ds(base_s + c * CH, CH)])
        return _

    lax.fori_loop(0, NCH, chunk_body, None)


@jax.jit
def kernel(mem, idx, val, sample_idx):
    mesh = plsc.VectorSubcoreMesh(
        core_axis_name="c", subcore_axis_name="s",
        num_cores=NC, num_subcores=NS)
    run = pl.kernel(
        _body,
        out_type=jax.ShapeDtypeStruct((B, D), jnp.float32),
        mesh=mesh,
        scratch_types=[
            pltpu.VMEM((B,), jnp.int32),
            pltpu.VMEM((RPT,), jnp.int32),
            pltpu.VMEM((SPW,), jnp.int32),
            pltpu.VMEM((SPW,), jnp.int32),
            pltpu.VMEM((CH,), jnp.int32),
            pltpu.VMEM((CH,), jnp.int32),
            pltpu.VMEM((CH,), jnp.float32),
            pltpu.VMEM((CH, D), jnp.float32),
            pltpu.VMEM((CH, D), jnp.float32),
            pltpu.HBM((NC * MPAD,), jnp.int32),
        ],
        compiler_params=pltpu.CompilerParams(
            needs_layout_passes=False, use_tc_tiling_on_sc=False),
    )
    return run(mem, idx, val, sample_idx)
